# Initial kernel scaffold; baseline (speedup 1.0000x reference)
import jax
import jax.numpy as jnp
from jax.experimental import pallas as pl


def kernel(num_x, x, edge_index, edge_type, params):
    raise NotImplementedError("write your pallas kernel here")


# speedup vs baseline: 12.3270x; 12.3270x over previous
import functools

import jax
import jax.numpy as jnp
from jax import lax
from jax.experimental import pallas as pl
from jax.experimental.pallas import tpu as pltpu
from jax.experimental.pallas import tpu_sc as plsc

N = 10000
E = 320000
R = 8
RP1 = 9
D = 128
NC = 2
NS = 16
CHUNK = 80

ROWS_PER_TILE = 624
ROWS_TAIL = N - NS * ROWS_PER_TILE
E_PER_TILE = E // (NC * NS)
E_PER_TILE_CNT = E // NS


def _scale_body(d_hbm, t_hbm, s_hbm,
                counts_sp, zbuf_v, qidx_v, ones_v, cbuf_v, didx_v, tidx_v,
                sbuf_v, sem):
    c = lax.axis_index("c")
    sid = lax.axis_index("s")

    ZW = 3200
    n_zchunks = N * R // ZW
    z16 = jnp.zeros((16,), jnp.float32)

    @pl.loop(0, ZW // 16)
    def _fill_zero(i):
        zbuf_v[pl.ds(i * 16, 16)] = z16

    @pl.loop(0, (n_zchunks + NS - 1) // NS)
    def _zero(k):
        chunk = sid + k * NS

        @pl.when(chunk < n_zchunks)
        def _():
            o = pl.multiple_of(chunk * ZW, 128)
            pltpu.sync_copy(zbuf_v.at[pl.ds(0, ZW)], counts_sp.at[pl.ds(o, ZW)])

    plsc.subcore_barrier()

    for j in range(CHUNK // 16):
        ones_v[pl.ds(j * 16, 16)] = jnp.full((16,), 1.0, jnp.float32)

    def _load_q(off):
        pltpu.sync_copy(d_hbm.at[pl.ds(off, CHUNK)], didx_v)
        pltpu.sync_copy(t_hbm.at[pl.ds(off, CHUNK)], tidx_v)
        for j in range(CHUNK // 16):
            sl = pl.ds(j * 16, 16)
            qidx_v[sl] = didx_v[sl] * R + tidx_v[sl]

    base_cnt = sid * E_PER_TILE_CNT

    @pl.loop(0, E_PER_TILE_CNT // CHUNK)
    def _count_chunk(ci):
        _load_q(pl.multiple_of(base_cnt + ci * CHUNK, 16))
        pltpu.sync_copy(ones_v, counts_sp.at[qidx_v], add=True)

    plsc.subcore_barrier()

    base = (c * NS + sid) * E_PER_TILE

    @pl.loop(0, E_PER_TILE // CHUNK)
    def _scale_chunk(ci):
        off = pl.multiple_of(base + ci * CHUNK, 16)
        _load_q(off)
        pltpu.async_copy(counts_sp.at[qidx_v], cbuf_v, sem).wait()
        for j in range(CHUNK // 16):
            sl = pl.ds(j * 16, 16)
            sbuf_v[sl] = 1.0 / jnp.maximum(cbuf_v[sl], 1.0)
        pltpu.sync_copy(sbuf_v, s_hbm.at[pl.ds(off, CHUNK)])


@functools.cache
def _get_scale_call():
    return functools.partial(
        pl.kernel,
        out_type=jax.ShapeDtypeStruct((E,), jnp.float32),
        mesh=plsc.VectorSubcoreMesh(core_axis_name="c", subcore_axis_name="s",
                                    num_cores=NC, num_subcores=NS),
        compiler_params=pltpu.CompilerParams(needs_layout_passes=False),
        scratch_types=[
            pltpu.VMEM_SHARED((N * R,), jnp.float32),
            pltpu.VMEM((3200,), jnp.float32),
            pltpu.VMEM((CHUNK,), jnp.int32),
            pltpu.VMEM((CHUNK,), jnp.float32),
            pltpu.VMEM((CHUNK,), jnp.float32),
            pltpu.VMEM((CHUNK,), jnp.int32),
            pltpu.VMEM((CHUNK,), jnp.int32),
            pltpu.VMEM((CHUNK,), jnp.float32),
            pltpu.SemaphoreType.DMA,
        ],
    )(_scale_body)


def _agg_body(y_hbm, g_hbm, d_hbm, s_hbm, zbig_hbm, part_hbm,
              acc_sp, rows_v, gidx_v, didx_v, s_v, sem):
    c = lax.axis_index("c")
    sid = lax.axis_index("s")

    rbase = pl.multiple_of(sid * ROWS_PER_TILE, 8)
    pltpu.sync_copy(zbig_hbm.at[pl.ds(rbase, ROWS_PER_TILE)],
                    acc_sp.at[pl.ds(rbase, ROWS_PER_TILE)])

    @pl.when(sid == NS - 1)
    def _zero_tail():
        pltpu.sync_copy(zbig_hbm.at[pl.ds(NS * ROWS_PER_TILE, ROWS_TAIL)],
                        acc_sp.at[pl.ds(NS * ROWS_PER_TILE, ROWS_TAIL)])

    plsc.subcore_barrier()

    base = (c * NS + sid) * E_PER_TILE

    @pl.loop(0, E_PER_TILE // CHUNK)
    def _edge_chunk(ci):
        off = pl.multiple_of(base + ci * CHUNK, 16)
        pltpu.sync_copy(g_hbm.at[pl.ds(off, CHUNK)], gidx_v)
        pltpu.sync_copy(d_hbm.at[pl.ds(off, CHUNK)], didx_v)
        pltpu.sync_copy(s_hbm.at[pl.ds(off, CHUNK)], s_v)
        pltpu.async_copy(y_hbm.at[gidx_v], rows_v, sem).wait()

        @pl.loop(0, CHUNK // 16)
        def _scale_rows(rg):
            sv16 = s_v[pl.ds(rg * 16, 16)]
            for k in range(16):
                r = rg * 16 + k
                svk = sv16[k]
                for j in range(D // 16):
                    sl = pl.ds(j * 16, 16)
                    rows_v[r, sl] = rows_v[r, sl] * svk

        pltpu.sync_copy(rows_v, acc_sp.at[didx_v], add=True)

    plsc.subcore_barrier()
    pltpu.sync_copy(acc_sp.at[pl.ds(rbase, ROWS_PER_TILE)],
                    part_hbm.at[c].at[pl.ds(rbase, ROWS_PER_TILE)])

    @pl.when(sid == NS - 1)
    def _copy_tail():
        pltpu.sync_copy(acc_sp.at[pl.ds(NS * ROWS_PER_TILE, ROWS_TAIL)],
                        part_hbm.at[c].at[pl.ds(NS * ROWS_PER_TILE, ROWS_TAIL)])


@functools.cache
def _get_agg_call():
    return functools.partial(
        pl.kernel,
        out_type=jax.ShapeDtypeStruct((NC, N, D), jnp.float32),
        mesh=plsc.VectorSubcoreMesh(core_axis_name="c", subcore_axis_name="s",
                                    num_cores=NC, num_subcores=NS),
        compiler_params=pltpu.CompilerParams(needs_layout_passes=False),
        scratch_types=[
            pltpu.VMEM_SHARED((N, D), jnp.float32),
            pltpu.VMEM((CHUNK, D), jnp.float32),
            pltpu.VMEM((CHUNK,), jnp.int32),
            pltpu.VMEM((CHUNK,), jnp.int32),
            pltpu.VMEM((CHUNK,), jnp.float32),
            pltpu.SemaphoreType.DMA,
        ],
    )(_agg_body)


MB = 400
WCOLS = RP1 * D


def _wprep_body(comp_ref, bases_ref, w_ref):
    w_ref[...] = jnp.dot(comp_ref[0], bases_ref[0],
                         preferred_element_type=jnp.float32)[None]


def _wprep(comp7, bases7):
    nb = comp7.shape[2]
    return pl.pallas_call(
        _wprep_body,
        grid=(7,),
        in_specs=[
            pl.BlockSpec((1, R, nb), lambda i: (i, 0, 0)),
            pl.BlockSpec((1, nb, D * 128), lambda i: (i, 0, 0)),
        ],
        out_specs=pl.BlockSpec((1, R, D * 128), lambda i: (i, 0, 0)),
        out_shape=jax.ShapeDtypeStruct((7, R, D * 128), jnp.float32),
    )(comp7, bases7)


def _mm_first_body(nx_ref, x_ref, nw_ref, nb_ref, na_ref, w_ref, y_ref):
    h = nx_ref[...] * nw_ref[...] + nb_ref[...]
    h = jnp.where(h >= 0, h, na_ref[...] * h) + x_ref[...]
    y_ref[...] = jnp.dot(h, w_ref[...], preferred_element_type=jnp.float32)


def _mm_first(num_x, x, nw, nb, na, wfull):
    return pl.pallas_call(
        _mm_first_body,
        grid=(N // MB,),
        in_specs=[
            pl.BlockSpec((MB, 1), lambda i: (i, 0)),
            pl.BlockSpec((MB, D), lambda i: (i, 0)),
            pl.BlockSpec((1, D), lambda i: (0, 0)),
            pl.BlockSpec((1, D), lambda i: (0, 0)),
            pl.BlockSpec((1, D), lambda i: (0, 0)),
            pl.BlockSpec((D, WCOLS), lambda i: (0, 0)),
        ],
        out_specs=pl.BlockSpec((MB, WCOLS), lambda i: (i, 0)),
        out_shape=jax.ShapeDtypeStruct((N, WCOLS), jnp.float32),
    )(num_x, x, nw, nb, na, wfull)


def _mm_mid_body(yprev_ref, a0_ref, a1_ref, b_ref, al_ref, w_ref, y_ref):
    h = yprev_ref[...] + b_ref[...] + a0_ref[...] + a1_ref[...]
    h = jnp.where(h >= 0, h, al_ref[...] * h)
    y_ref[...] = jnp.dot(h, w_ref[...], preferred_element_type=jnp.float32)


def _mm_mid(yprev, a0, a1, b, al, wfull):
    return pl.pallas_call(
        _mm_mid_body,
        grid=(N // MB,),
        in_specs=[
            pl.BlockSpec((MB, D), lambda i: (i, R)),
            pl.BlockSpec((MB, D), lambda i: (i, 0)),
            pl.BlockSpec((MB, D), lambda i: (i, 0)),
            pl.BlockSpec((1, D), lambda i: (0, 0)),
            pl.BlockSpec((1, D), lambda i: (0, 0)),
            pl.BlockSpec((D, WCOLS), lambda i: (0, 0)),
        ],
        out_specs=pl.BlockSpec((MB, WCOLS), lambda i: (i, 0)),
        out_shape=jax.ShapeDtypeStruct((N, WCOLS), jnp.float32),
    )(yprev, a0, a1, b, al, wfull)


def _final_body(yprev_ref, a0_ref, a1_ref, b_ref, o_ref):
    z = yprev_ref[...] + b_ref[...] + a0_ref[...] + a1_ref[...]
    col = lax.broadcasted_iota(jnp.int32, z.shape, 1)
    valid = col < 3
    m = jnp.max(jnp.where(valid, z, -jnp.inf), axis=1, keepdims=True)
    e = jnp.where(valid, jnp.exp(z - m), 0.0)
    o_ref[...] = z - m - jnp.log(jnp.sum(e, axis=1, keepdims=True))


def _final(yprev, a0, a1, b):
    return pl.pallas_call(
        _final_body,
        grid=(N // MB,),
        in_specs=[
            pl.BlockSpec((MB, D), lambda i: (i, R)),
            pl.BlockSpec((MB, D), lambda i: (i, 0)),
            pl.BlockSpec((MB, D), lambda i: (i, 0)),
            pl.BlockSpec((1, D), lambda i: (0, 0)),
        ],
        out_specs=pl.BlockSpec((MB, D), lambda i: (i, 0)),
        out_shape=jax.ShapeDtypeStruct((N, D), jnp.float32),
    )(yprev, a0, a1, b)


def kernel(num_x, x, edge_index, edge_type, params):
    src = edge_index[0].astype(jnp.int32)
    dst = edge_index[1].astype(jnp.int32)
    t = edge_type.astype(jnp.int32)
    g = src * RP1 + t

    zbig = jnp.zeros((N, D), jnp.float32)

    s = _get_scale_call()(dst, t)

    bases_l = []
    comp_l = []
    wfulls = []
    for l in range(7):
        pp = params['conv%d' % l]
        b = pp['bases']
        dout = b.shape[2]
        if dout < 128:
            b = jnp.pad(b, ((0, 0), (0, 0), (0, 128 - dout)))
        bases_l.append(b.reshape(b.shape[0], D * 128))
        comp_l.append(pp['comp'])
    w7 = _wprep(jnp.stack(comp_l), jnp.stack(bases_l))
    for l in range(7):
        pp = params['conv%d' % l]
        dout = pp['root'].shape[1]
        rootp = pp['root']
        if dout < 128:
            rootp = jnp.pad(rootp, ((0, 0), (0, 128 - dout)))
        wrel = jnp.transpose(w7[l].reshape(R, D, 128), (1, 0, 2)).reshape(D, R * 128)
        wfulls.append(jnp.concatenate([wrel, rootp], axis=1))

    def pad128(v):
        return jnp.pad(v, (0, 128 - v.shape[0]))[None] if v.shape[0] < 128 else v[None]

    y = _mm_first(num_x, x,
                  params['num_lin_w'], params['num_lin_b'][None],
                  params['prelu_lin'][None], wfulls[0])
    for l in range(1, 7):
        part = _get_agg_call()(y.reshape(N * RP1, D), g, dst, s, zbig)
        y = _mm_mid(y, part[0], part[1],
                    pad128(params['conv%d' % (l - 1)]['bias']),
                    pad128(params['prelu%d' % (l - 1)]), wfulls[l])
    part = _get_agg_call()(y.reshape(N * RP1, D), g, dst, s, zbig)
    out = _final(y, part[0], part[1], pad128(params['conv6']['bias']))
    return out[:, :3]


# speedup vs baseline: 25.9439x; 2.1047x over previous
import functools

import jax
import jax.numpy as jnp
from jax import lax
from jax.experimental import pallas as pl
from jax.experimental.pallas import tpu as pltpu
from jax.experimental.pallas import tpu_sc as plsc

N = 10000
E = 320000
R = 8
RP1 = 9
D = 128
NC = 2
NS = 16
CHUNK = 80

ROWS_PER_TILE = 624
ROWS_TAIL = N - NS * ROWS_PER_TILE
E_PER_TILE = E // (NC * NS)
E_PER_TILE_CNT = E // NS


def _scale_body(d_hbm, t_hbm, s_hbm,
                counts_sp, zbuf_v, qidx_v, ones_v, cbuf_v, didx_v, tidx_v,
                sbuf_v, sem):
    c = lax.axis_index("c")
    sid = lax.axis_index("s")

    ZW = 3200
    n_zchunks = N * R // ZW
    z16 = jnp.zeros((16,), jnp.float32)

    @pl.loop(0, ZW // 16)
    def _fill_zero(i):
        zbuf_v[pl.ds(i * 16, 16)] = z16

    @pl.loop(0, (n_zchunks + NS - 1) // NS)
    def _zero(k):
        chunk = sid + k * NS

        @pl.when(chunk < n_zchunks)
        def _():
            o = pl.multiple_of(chunk * ZW, 128)
            pltpu.sync_copy(zbuf_v.at[pl.ds(0, ZW)], counts_sp.at[pl.ds(o, ZW)])

    plsc.subcore_barrier()

    for j in range(CHUNK // 16):
        ones_v[pl.ds(j * 16, 16)] = jnp.full((16,), 1.0, jnp.float32)

    def _load_q(off):
        pltpu.sync_copy(d_hbm.at[pl.ds(off, CHUNK)], didx_v)
        pltpu.sync_copy(t_hbm.at[pl.ds(off, CHUNK)], tidx_v)
        for j in range(CHUNK // 16):
            sl = pl.ds(j * 16, 16)
            qidx_v[sl] = didx_v[sl] * R + tidx_v[sl]

    base_cnt = sid * E_PER_TILE_CNT

    @pl.loop(0, E_PER_TILE_CNT // CHUNK)
    def _count_chunk(ci):
        _load_q(pl.multiple_of(base_cnt + ci * CHUNK, 16))
        pltpu.sync_copy(ones_v, counts_sp.at[qidx_v], add=True)

    plsc.subcore_barrier()

    base = (c * NS + sid) * E_PER_TILE

    @pl.loop(0, E_PER_TILE // CHUNK)
    def _scale_chunk(ci):
        off = pl.multiple_of(base + ci * CHUNK, 16)
        _load_q(off)
        pltpu.async_copy(counts_sp.at[qidx_v], cbuf_v, sem).wait()
        for j in range(CHUNK // 16):
            sl = pl.ds(j * 16, 16)
            sbuf_v[sl] = 1.0 / jnp.maximum(cbuf_v[sl], 1.0)
        pltpu.sync_copy(sbuf_v, s_hbm.at[pl.ds(off, CHUNK)])


@functools.cache
def _get_scale_call():
    return functools.partial(
        pl.kernel,
        out_type=jax.ShapeDtypeStruct((E,), jnp.float32),
        mesh=plsc.VectorSubcoreMesh(core_axis_name="c", subcore_axis_name="s",
                                    num_cores=NC, num_subcores=NS),
        compiler_params=pltpu.CompilerParams(needs_layout_passes=False),
        scratch_types=[
            pltpu.VMEM_SHARED((N * R,), jnp.float32),
            pltpu.VMEM((3200,), jnp.float32),
            pltpu.VMEM((CHUNK,), jnp.int32),
            pltpu.VMEM((CHUNK,), jnp.float32),
            pltpu.VMEM((CHUNK,), jnp.float32),
            pltpu.VMEM((CHUNK,), jnp.int32),
            pltpu.VMEM((CHUNK,), jnp.int32),
            pltpu.VMEM((CHUNK,), jnp.float32),
            pltpu.SemaphoreType.DMA,
        ],
    )(_scale_body)


NBUF = 4
NCH = E_PER_TILE // CHUNK


def _agg_body(y_hbm, g_hbm, d_hbm, s_hbm, zbig_hbm, part_hbm,
              acc_sp, rows0, rows1, rows2, rows3,
              gv0, gv1, gv2, gv3, dv0, dv1, dv2, dv3, sv0, sv1, sv2, sv3,
              is0, is1, is2, is3, gs0, gs1, gs2, gs3, ss0, ss1, ss2, ss3):
    c = lax.axis_index("c")
    sid = lax.axis_index("s")
    rows = (rows0, rows1, rows2, rows3)
    gvs = (gv0, gv1, gv2, gv3)
    dvs = (dv0, dv1, dv2, dv3)
    svs = (sv0, sv1, sv2, sv3)
    isems = (is0, is1, is2, is3)
    gsems = (gs0, gs1, gs2, gs3)
    ssems = (ss0, ss1, ss2, ss3)

    rbase = pl.multiple_of(sid * ROWS_PER_TILE, 8)
    pltpu.sync_copy(zbig_hbm.at[pl.ds(rbase, ROWS_PER_TILE)],
                    acc_sp.at[pl.ds(rbase, ROWS_PER_TILE)])

    @pl.when(sid == NS - 1)
    def _zero_tail():
        pltpu.sync_copy(zbig_hbm.at[pl.ds(NS * ROWS_PER_TILE, ROWS_TAIL)],
                        acc_sp.at[pl.ds(NS * ROWS_PER_TILE, ROWS_TAIL)])

    plsc.subcore_barrier()

    cbase = (c * NS + sid) * NCH

    def fire_idx(b, ci):
        off = pl.multiple_of((cbase + ci) * CHUNK, 16)
        pltpu.async_copy(g_hbm.at[pl.ds(off, CHUNK)], gvs[b], isems[b])
        pltpu.async_copy(d_hbm.at[pl.ds(off, CHUNK)], dvs[b], isems[b])
        pltpu.async_copy(s_hbm.at[pl.ds(off, CHUNK)], svs[b], isems[b])

    def wait_idx(b, ci):
        off = pl.multiple_of((cbase + ci) * CHUNK, 16)
        pltpu.make_async_copy(g_hbm.at[pl.ds(off, CHUNK)], gvs[b],
                              isems[b]).wait()
        pltpu.make_async_copy(d_hbm.at[pl.ds(off, CHUNK)], dvs[b],
                              isems[b]).wait()
        pltpu.make_async_copy(s_hbm.at[pl.ds(off, CHUNK)], svs[b],
                              isems[b]).wait()

    def fire_gather(b):
        pltpu.async_copy(y_hbm.at[gvs[b]], rows[b], gsems[b])

    def wait_gather(b):
        pltpu.make_async_copy(y_hbm.at[gvs[b]], rows[b], gsems[b]).wait()

    def fire_scatter(b):
        pltpu.async_copy(rows[b], acc_sp.at[dvs[b]], ssems[b], add=True)

    def wait_scatter(b):
        pltpu.make_async_copy(rows[b], acc_sp.at[dvs[b]], ssems[b]).wait()

    def scale(b):
        @pl.loop(0, CHUNK // 16)
        def _scale_rows(rg):
            sv16 = svs[b][pl.ds(rg * 16, 16)]
            for kk in range(16):
                r = rg * 16 + kk
                svk = sv16[kk]
                for j in range(D // 16):
                    sl = pl.ds(j * 16, 16)
                    rows[b][r, sl] = rows[b][r, sl] * svk

    def visit(ci, boff):
        b = boff
        bn = (boff + 1) % NBUF
        b2 = (boff + 2) % NBUF

        @pl.when(ci >= 2)
        def _():
            wait_scatter(b2)

        @pl.when(ci + 2 < NCH)
        def _():
            fire_idx(b2, ci + 2)

        @pl.when(ci + 1 < NCH)
        def _():
            wait_idx(bn, ci + 1)
            fire_gather(bn)

        wait_gather(b)
        scale(b)
        fire_scatter(b)

    fire_idx(0, 0)
    fire_idx(1, 1)
    wait_idx(0, 0)
    fire_gather(0)

    @pl.loop(0, NCH // NBUF)
    def _ring(it):
        for boff in range(NBUF):
            visit(it * NBUF + boff, boff)

    visit(NCH - 1, (NCH - 1) % NBUF)
    wait_scatter((NCH - 2) % NBUF)
    wait_scatter((NCH - 1) % NBUF)

    plsc.subcore_barrier()
    pltpu.sync_copy(acc_sp.at[pl.ds(rbase, ROWS_PER_TILE)],
                    part_hbm.at[c].at[pl.ds(rbase, ROWS_PER_TILE)])

    @pl.when(sid == NS - 1)
    def _copy_tail():
        pltpu.sync_copy(acc_sp.at[pl.ds(NS * ROWS_PER_TILE, ROWS_TAIL)],
                        part_hbm.at[c].at[pl.ds(NS * ROWS_PER_TILE, ROWS_TAIL)])


@functools.cache
def _get_agg_call():
    return functools.partial(
        pl.kernel,
        out_type=jax.ShapeDtypeStruct((NC, N, D), jnp.float32),
        mesh=plsc.VectorSubcoreMesh(core_axis_name="c", subcore_axis_name="s",
                                    num_cores=NC, num_subcores=NS),
        compiler_params=pltpu.CompilerParams(needs_layout_passes=False),
        scratch_types=(
            [pltpu.VMEM_SHARED((N, D), jnp.float32)]
            + [pltpu.VMEM((CHUNK, D), jnp.float32) for _ in range(NBUF)]
            + [pltpu.VMEM((CHUNK,), jnp.int32) for _ in range(2 * NBUF)]
            + [pltpu.VMEM((CHUNK,), jnp.float32) for _ in range(NBUF)]
            + [pltpu.SemaphoreType.DMA for _ in range(3 * NBUF)]
        ),
    )(_agg_body)


MB = 400
WCOLS = RP1 * D


def _wprep_body(comp_ref, bases_ref, w_ref):
    w_ref[...] = jnp.dot(comp_ref[0], bases_ref[0],
                         preferred_element_type=jnp.float32)[None]


def _wprep(comp7, bases7):
    nb = comp7.shape[2]
    return pl.pallas_call(
        _wprep_body,
        grid=(7,),
        in_specs=[
            pl.BlockSpec((1, R, nb), lambda i: (i, 0, 0)),
            pl.BlockSpec((1, nb, D * 128), lambda i: (i, 0, 0)),
        ],
        out_specs=pl.BlockSpec((1, R, D * 128), lambda i: (i, 0, 0)),
        out_shape=jax.ShapeDtypeStruct((7, R, D * 128), jnp.float32),
    )(comp7, bases7)


def _mm_first_body(nx_ref, x_ref, nw_ref, nb_ref, na_ref, w_ref, y_ref):
    h = nx_ref[...] * nw_ref[...] + nb_ref[...]
    h = jnp.where(h >= 0, h, na_ref[...] * h) + x_ref[...]
    y_ref[...] = jnp.dot(h, w_ref[...], preferred_element_type=jnp.float32)


def _mm_first(num_x, x, nw, nb, na, wfull):
    return pl.pallas_call(
        _mm_first_body,
        grid=(N // MB,),
        in_specs=[
            pl.BlockSpec((MB, 1), lambda i: (i, 0)),
            pl.BlockSpec((MB, D), lambda i: (i, 0)),
            pl.BlockSpec((1, D), lambda i: (0, 0)),
            pl.BlockSpec((1, D), lambda i: (0, 0)),
            pl.BlockSpec((1, D), lambda i: (0, 0)),
            pl.BlockSpec((D, WCOLS), lambda i: (0, 0)),
        ],
        out_specs=pl.BlockSpec((MB, WCOLS), lambda i: (i, 0)),
        out_shape=jax.ShapeDtypeStruct((N, WCOLS), jnp.float32),
    )(num_x, x, nw, nb, na, wfull)


def _mm_mid_body(yprev_ref, a0_ref, a1_ref, b_ref, al_ref, w_ref, y_ref):
    h = yprev_ref[...] + b_ref[...] + a0_ref[...] + a1_ref[...]
    h = jnp.where(h >= 0, h, al_ref[...] * h)
    y_ref[...] = jnp.dot(h, w_ref[...], preferred_element_type=jnp.float32)


def _mm_mid(yprev, a0, a1, b, al, wfull):
    return pl.pallas_call(
        _mm_mid_body,
        grid=(N // MB,),
        in_specs=[
            pl.BlockSpec((MB, D), lambda i: (i, R)),
            pl.BlockSpec((MB, D), lambda i: (i, 0)),
            pl.BlockSpec((MB, D), lambda i: (i, 0)),
            pl.BlockSpec((1, D), lambda i: (0, 0)),
            pl.BlockSpec((1, D), lambda i: (0, 0)),
            pl.BlockSpec((D, WCOLS), lambda i: (0, 0)),
        ],
        out_specs=pl.BlockSpec((MB, WCOLS), lambda i: (i, 0)),
        out_shape=jax.ShapeDtypeStruct((N, WCOLS), jnp.float32),
    )(yprev, a0, a1, b, al, wfull)


def _final_body(yprev_ref, a0_ref, a1_ref, b_ref, o_ref):
    z = yprev_ref[...] + b_ref[...] + a0_ref[...] + a1_ref[...]
    col = lax.broadcasted_iota(jnp.int32, z.shape, 1)
    valid = col < 3
    m = jnp.max(jnp.where(valid, z, -jnp.inf), axis=1, keepdims=True)
    e = jnp.where(valid, jnp.exp(z - m), 0.0)
    o_ref[...] = z - m - jnp.log(jnp.sum(e, axis=1, keepdims=True))


def _final(yprev, a0, a1, b):
    return pl.pallas_call(
        _final_body,
        grid=(N // MB,),
        in_specs=[
            pl.BlockSpec((MB, D), lambda i: (i, R)),
            pl.BlockSpec((MB, D), lambda i: (i, 0)),
            pl.BlockSpec((MB, D), lambda i: (i, 0)),
            pl.BlockSpec((1, D), lambda i: (0, 0)),
        ],
        out_specs=pl.BlockSpec((MB, D), lambda i: (i, 0)),
        out_shape=jax.ShapeDtypeStruct((N, D), jnp.float32),
    )(yprev, a0, a1, b)


def kernel(num_x, x, edge_index, edge_type, params):
    src = edge_index[0].astype(jnp.int32)
    dst = edge_index[1].astype(jnp.int32)
    t = edge_type.astype(jnp.int32)
    g = src * RP1 + t

    zbig = jnp.zeros((N, D), jnp.float32)

    s = _get_scale_call()(dst, t)

    bases_l = []
    comp_l = []
    wfulls = []
    for l in range(7):
        pp = params['conv%d' % l]
        b = pp['bases']
        dout = b.shape[2]
        if dout < 128:
            b = jnp.pad(b, ((0, 0), (0, 0), (0, 128 - dout)))
        bases_l.append(b.reshape(b.shape[0], D * 128))
        comp_l.append(pp['comp'])
    w7 = _wprep(jnp.stack(comp_l), jnp.stack(bases_l))
    for l in range(7):
        pp = params['conv%d' % l]
        dout = pp['root'].shape[1]
        rootp = pp['root']
        if dout < 128:
            rootp = jnp.pad(rootp, ((0, 0), (0, 128 - dout)))
        wrel = jnp.transpose(w7[l].reshape(R, D, 128), (1, 0, 2)).reshape(D, R * 128)
        wfulls.append(jnp.concatenate([wrel, rootp], axis=1))

    def pad128(v):
        return jnp.pad(v, (0, 128 - v.shape[0]))[None] if v.shape[0] < 128 else v[None]

    y = _mm_first(num_x, x,
                  params['num_lin_w'], params['num_lin_b'][None],
                  params['prelu_lin'][None], wfulls[0])
    for l in range(1, 7):
        part = _get_agg_call()(y.reshape(N * RP1, D), g, dst, s, zbig)
        y = _mm_mid(y, part[0], part[1],
                    pad128(params['conv%d' % (l - 1)]['bias']),
                    pad128(params['prelu%d' % (l - 1)]), wfulls[l])
    part = _get_agg_call()(y.reshape(N * RP1, D), g, dst, s, zbig)
    out = _final(y, part[0], part[1], pad128(params['conv6']['bias']))
    return out[:, :3]


# speedup vs baseline: 29.9577x; 1.1547x over previous
import functools

import jax
import jax.numpy as jnp
from jax import lax
from jax.experimental import pallas as pl
from jax.experimental.pallas import tpu as pltpu
from jax.experimental.pallas import tpu_sc as plsc

N = 10000
E = 320000
R = 8
RP1 = 9
D = 128
NC = 2
NS = 16
CHUNK = 80

ROWS_PER_TILE = 624
ROWS_TAIL = N - NS * ROWS_PER_TILE
E_PER_TILE = E // (NC * NS)
E_PER_TILE_CNT = E // NS


def _scale_body(d_hbm, t_hbm, s_hbm,
                counts_sp, zbuf_v, ones_v,
                qx0, qx1, cb0, cb1, dv0, dv1, tv0, tv1, sb0, sb1,
                is0, is1, ss0, ss1, gs0, gs1, os0, os1):
    c = lax.axis_index("c")
    sid = lax.axis_index("s")
    qxs = (qx0, qx1)
    cbs = (cb0, cb1)
    dvs = (dv0, dv1)
    tvs = (tv0, tv1)
    sbs = (sb0, sb1)
    isems = (is0, is1)
    ssems = (ss0, ss1)
    gsems = (gs0, gs1)
    osems = (os0, os1)

    ZW = 3200
    n_zchunks = N * R // ZW
    z16 = jnp.zeros((16,), jnp.float32)

    @pl.loop(0, ZW // 16)
    def _fill_zero(i):
        zbuf_v[pl.ds(i * 16, 16)] = z16

    @pl.loop(0, (n_zchunks + NS - 1) // NS)
    def _zero(k):
        chunk = sid + k * NS

        @pl.when(chunk < n_zchunks)
        def _():
            o = pl.multiple_of(chunk * ZW, 128)
            pltpu.sync_copy(zbuf_v.at[pl.ds(0, ZW)], counts_sp.at[pl.ds(o, ZW)])

    plsc.subcore_barrier()

    for j in range(CHUNK // 16):
        ones_v[pl.ds(j * 16, 16)] = jnp.full((16,), 1.0, jnp.float32)

    def fire_idx(b, base, ci):
        off = pl.multiple_of(base + ci * CHUNK, 16)
        pltpu.async_copy(d_hbm.at[pl.ds(off, CHUNK)], dvs[b], isems[b])
        pltpu.async_copy(t_hbm.at[pl.ds(off, CHUNK)], tvs[b], isems[b])

    def wait_idx(b, base, ci):
        off = pl.multiple_of(base + ci * CHUNK, 16)
        pltpu.make_async_copy(d_hbm.at[pl.ds(off, CHUNK)], dvs[b],
                              isems[b]).wait()
        pltpu.make_async_copy(t_hbm.at[pl.ds(off, CHUNK)], tvs[b],
                              isems[b]).wait()

    def compute_q(b):
        for j in range(CHUNK // 16):
            sl = pl.ds(j * 16, 16)
            qxs[b][sl] = dvs[b][sl] * R + tvs[b][sl]

    base_cnt = sid * E_PER_TILE_CNT
    NCHT = E_PER_TILE_CNT // CHUNK

    def cvisit(ci, b):
        bn = 1 - b

        @pl.when(ci >= 2)
        def _():
            pltpu.make_async_copy(ones_v, counts_sp.at[qxs[b]],
                                  ssems[b]).wait()

        @pl.when(ci + 1 < NCHT)
        def _():
            fire_idx(bn, base_cnt, ci + 1)

        wait_idx(b, base_cnt, ci)
        compute_q(b)
        pltpu.async_copy(ones_v, counts_sp.at[qxs[b]], ssems[b], add=True)

    fire_idx(0, base_cnt, 0)

    @pl.loop(0, NCHT // 2)
    def _count_ring(it):
        for b in range(2):
            cvisit(it * 2 + b, b)

    pltpu.make_async_copy(ones_v, counts_sp.at[qxs[0]], ssems[0]).wait()
    pltpu.make_async_copy(ones_v, counts_sp.at[qxs[1]], ssems[1]).wait()
    plsc.subcore_barrier()

    base = (c * NS + sid) * E_PER_TILE
    NCHS = E_PER_TILE // CHUNK

    def out_desc(b, ci):
        off = pl.multiple_of(base + ci * CHUNK, 16)
        return pltpu.make_async_copy(sbs[b], s_hbm.at[pl.ds(off, CHUNK)],
                                     osems[b])

    def svisit(ci, b):
        bn = 1 - b

        @pl.when(ci + 1 < NCHS)
        def _():
            fire_idx(bn, base, ci + 1)

        pltpu.make_async_copy(counts_sp.at[qxs[b]], cbs[b], gsems[b]).wait()

        @pl.when(ci >= 2)
        def _():
            out_desc(b, ci - 2).wait()

        for j in range(CHUNK // 16):
            sl = pl.ds(j * 16, 16)
            sbs[b][sl] = 1.0 / jnp.maximum(cbs[b][sl], 1.0)
        off = pl.multiple_of(base + ci * CHUNK, 16)
        pltpu.async_copy(sbs[b], s_hbm.at[pl.ds(off, CHUNK)], osems[b])

        @pl.when(ci + 1 < NCHS)
        def _():
            wait_idx(bn, base, ci + 1)
            compute_q(bn)
            pltpu.async_copy(counts_sp.at[qxs[bn]], cbs[bn], gsems[bn])

    fire_idx(0, base, 0)
    wait_idx(0, base, 0)
    compute_q(0)
    pltpu.async_copy(counts_sp.at[qxs[0]], cbs[0], gsems[0])

    @pl.loop(0, NCHS // 2)
    def _scale_ring(it):
        for b in range(2):
            svisit(it * 2 + b, b)

    svisit(NCHS - 1, (NCHS - 1) % 2)
    out_desc((NCHS - 2) % 2, NCHS - 2).wait()
    out_desc((NCHS - 1) % 2, NCHS - 1).wait()


@functools.cache
def _get_scale_call():
    return functools.partial(
        pl.kernel,
        out_type=jax.ShapeDtypeStruct((E,), jnp.float32),
        mesh=plsc.VectorSubcoreMesh(core_axis_name="c", subcore_axis_name="s",
                                    num_cores=NC, num_subcores=NS),
        compiler_params=pltpu.CompilerParams(needs_layout_passes=False),
        scratch_types=(
            [pltpu.VMEM_SHARED((N * R,), jnp.float32),
             pltpu.VMEM((3200,), jnp.float32),
             pltpu.VMEM((CHUNK,), jnp.float32)]
            + [pltpu.VMEM((CHUNK,), jnp.int32) for _ in range(2)]
            + [pltpu.VMEM((CHUNK,), jnp.float32) for _ in range(2)]
            + [pltpu.VMEM((CHUNK,), jnp.int32) for _ in range(4)]
            + [pltpu.VMEM((CHUNK,), jnp.float32) for _ in range(2)]
            + [pltpu.SemaphoreType.DMA for _ in range(8)]
        ),
    )(_scale_body)


NBUF = 4
NCH = E_PER_TILE // CHUNK


def _agg_body(y_hbm, g_hbm, d_hbm, s_hbm, zbig_hbm, part_hbm,
              acc_sp, rows0, rows1, rows2, rows3,
              gv0, gv1, gv2, gv3, dv0, dv1, dv2, dv3, sv0, sv1, sv2, sv3,
              is0, is1, is2, is3, gs0, gs1, gs2, gs3, ss0, ss1, ss2, ss3):
    c = lax.axis_index("c")
    sid = lax.axis_index("s")
    rows = (rows0, rows1, rows2, rows3)
    gvs = (gv0, gv1, gv2, gv3)
    dvs = (dv0, dv1, dv2, dv3)
    svs = (sv0, sv1, sv2, sv3)
    isems = (is0, is1, is2, is3)
    gsems = (gs0, gs1, gs2, gs3)
    ssems = (ss0, ss1, ss2, ss3)

    rbase = pl.multiple_of(sid * ROWS_PER_TILE, 8)
    pltpu.sync_copy(zbig_hbm.at[pl.ds(rbase, ROWS_PER_TILE)],
                    acc_sp.at[pl.ds(rbase, ROWS_PER_TILE)])

    @pl.when(sid == NS - 1)
    def _zero_tail():
        pltpu.sync_copy(zbig_hbm.at[pl.ds(NS * ROWS_PER_TILE, ROWS_TAIL)],
                        acc_sp.at[pl.ds(NS * ROWS_PER_TILE, ROWS_TAIL)])

    plsc.subcore_barrier()

    cbase = (c * NS + sid) * NCH

    def fire_idx(b, ci):
        off = pl.multiple_of((cbase + ci) * CHUNK, 16)
        pltpu.async_copy(g_hbm.at[pl.ds(off, CHUNK)], gvs[b], isems[b])
        pltpu.async_copy(d_hbm.at[pl.ds(off, CHUNK)], dvs[b], isems[b])
        pltpu.async_copy(s_hbm.at[pl.ds(off, CHUNK)], svs[b], isems[b])

    def wait_idx(b, ci):
        off = pl.multiple_of((cbase + ci) * CHUNK, 16)
        pltpu.make_async_copy(g_hbm.at[pl.ds(off, CHUNK)], gvs[b],
                              isems[b]).wait()
        pltpu.make_async_copy(d_hbm.at[pl.ds(off, CHUNK)], dvs[b],
                              isems[b]).wait()
        pltpu.make_async_copy(s_hbm.at[pl.ds(off, CHUNK)], svs[b],
                              isems[b]).wait()

    def fire_gather(b):
        pltpu.async_copy(y_hbm.at[gvs[b]], rows[b], gsems[b])

    def wait_gather(b):
        pltpu.make_async_copy(y_hbm.at[gvs[b]], rows[b], gsems[b]).wait()

    def fire_scatter(b):
        pltpu.async_copy(rows[b], acc_sp.at[dvs[b]], ssems[b], add=True)

    def wait_scatter(b):
        pltpu.make_async_copy(rows[b], acc_sp.at[dvs[b]], ssems[b]).wait()

    def scale(b):
        @pl.loop(0, CHUNK // 16)
        def _scale_rows(rg):
            sv16 = svs[b][pl.ds(rg * 16, 16)]
            for kk in range(16):
                r = rg * 16 + kk
                svk = sv16[kk]
                for j in range(D // 16):
                    sl = pl.ds(j * 16, 16)
                    rows[b][r, sl] = rows[b][r, sl] * svk

    def visit(ci, boff):
        b = boff
        bn = (boff + 1) % NBUF
        b2 = (boff + 2) % NBUF

        @pl.when(ci >= 2)
        def _():
            wait_scatter(b2)

        @pl.when(ci + 2 < NCH)
        def _():
            fire_idx(b2, ci + 2)

        @pl.when(ci + 1 < NCH)
        def _():
            wait_idx(bn, ci + 1)
            fire_gather(bn)

        wait_gather(b)
        scale(b)
        fire_scatter(b)

    fire_idx(0, 0)
    fire_idx(1, 1)
    wait_idx(0, 0)
    fire_gather(0)

    @pl.loop(0, NCH // NBUF)
    def _ring(it):
        for boff in range(NBUF):
            visit(it * NBUF + boff, boff)

    visit(NCH - 1, (NCH - 1) % NBUF)
    wait_scatter((NCH - 2) % NBUF)
    wait_scatter((NCH - 1) % NBUF)

    plsc.subcore_barrier()
    pltpu.sync_copy(acc_sp.at[pl.ds(rbase, ROWS_PER_TILE)],
                    part_hbm.at[c].at[pl.ds(rbase, ROWS_PER_TILE)])

    @pl.when(sid == NS - 1)
    def _copy_tail():
        pltpu.sync_copy(acc_sp.at[pl.ds(NS * ROWS_PER_TILE, ROWS_TAIL)],
                        part_hbm.at[c].at[pl.ds(NS * ROWS_PER_TILE, ROWS_TAIL)])


@functools.cache
def _get_agg_call():
    return functools.partial(
        pl.kernel,
        out_type=jax.ShapeDtypeStruct((NC, N, D), jnp.float32),
        mesh=plsc.VectorSubcoreMesh(core_axis_name="c", subcore_axis_name="s",
                                    num_cores=NC, num_subcores=NS),
        compiler_params=pltpu.CompilerParams(needs_layout_passes=False),
        scratch_types=(
            [pltpu.VMEM_SHARED((N, D), jnp.float32)]
            + [pltpu.VMEM((CHUNK, D), jnp.float32) for _ in range(NBUF)]
            + [pltpu.VMEM((CHUNK,), jnp.int32) for _ in range(2 * NBUF)]
            + [pltpu.VMEM((CHUNK,), jnp.float32) for _ in range(NBUF)]
            + [pltpu.SemaphoreType.DMA for _ in range(3 * NBUF)]
        ),
    )(_agg_body)


MB = 400
WCOLS = RP1 * D


def _wprep_body(comp_ref, bases_ref, w_ref):
    w_ref[...] = jnp.dot(comp_ref[0], bases_ref[0],
                         preferred_element_type=jnp.float32)[None]


def _wprep(comp7, bases7):
    nb = comp7.shape[2]
    return pl.pallas_call(
        _wprep_body,
        grid=(7,),
        in_specs=[
            pl.BlockSpec((1, R, nb), lambda i: (i, 0, 0)),
            pl.BlockSpec((1, nb, D * 128), lambda i: (i, 0, 0)),
        ],
        out_specs=pl.BlockSpec((1, R, D * 128), lambda i: (i, 0, 0)),
        out_shape=jax.ShapeDtypeStruct((7, R, D * 128), jnp.float32),
    )(comp7, bases7)


def _mm_first_body(nx_ref, x_ref, nw_ref, nb_ref, na_ref, w_ref, y_ref):
    h = nx_ref[...] * nw_ref[...] + nb_ref[...]
    h = jnp.where(h >= 0, h, na_ref[...] * h) + x_ref[...]
    y_ref[...] = jnp.dot(h, w_ref[...], preferred_element_type=jnp.float32)


def _mm_first(num_x, x, nw, nb, na, wfull):
    return pl.pallas_call(
        _mm_first_body,
        grid=(N // MB,),
        in_specs=[
            pl.BlockSpec((MB, 1), lambda i: (i, 0)),
            pl.BlockSpec((MB, D), lambda i: (i, 0)),
            pl.BlockSpec((1, D), lambda i: (0, 0)),
            pl.BlockSpec((1, D), lambda i: (0, 0)),
            pl.BlockSpec((1, D), lambda i: (0, 0)),
            pl.BlockSpec((D, WCOLS), lambda i: (0, 0)),
        ],
        out_specs=pl.BlockSpec((MB, WCOLS), lambda i: (i, 0)),
        out_shape=jax.ShapeDtypeStruct((N, WCOLS), jnp.float32),
    )(num_x, x, nw, nb, na, wfull)


def _mm_mid_body(yprev_ref, a0_ref, a1_ref, b_ref, al_ref, w_ref, y_ref):
    h = yprev_ref[...] + b_ref[...] + a0_ref[...] + a1_ref[...]
    h = jnp.where(h >= 0, h, al_ref[...] * h)
    y_ref[...] = jnp.dot(h, w_ref[...], preferred_element_type=jnp.float32)


def _mm_mid(yprev, a0, a1, b, al, wfull):
    return pl.pallas_call(
        _mm_mid_body,
        grid=(N // MB,),
        in_specs=[
            pl.BlockSpec((MB, D), lambda i: (i, R)),
            pl.BlockSpec((MB, D), lambda i: (i, 0)),
            pl.BlockSpec((MB, D), lambda i: (i, 0)),
            pl.BlockSpec((1, D), lambda i: (0, 0)),
            pl.BlockSpec((1, D), lambda i: (0, 0)),
            pl.BlockSpec((D, WCOLS), lambda i: (0, 0)),
        ],
        out_specs=pl.BlockSpec((MB, WCOLS), lambda i: (i, 0)),
        out_shape=jax.ShapeDtypeStruct((N, WCOLS), jnp.float32),
    )(yprev, a0, a1, b, al, wfull)


def _final_body(yprev_ref, a0_ref, a1_ref, b_ref, o_ref):
    z = yprev_ref[...] + b_ref[...] + a0_ref[...] + a1_ref[...]
    col = lax.broadcasted_iota(jnp.int32, z.shape, 1)
    valid = col < 3
    m = jnp.max(jnp.where(valid, z, -jnp.inf), axis=1, keepdims=True)
    e = jnp.where(valid, jnp.exp(z - m), 0.0)
    o_ref[...] = z - m - jnp.log(jnp.sum(e, axis=1, keepdims=True))


def _final(yprev, a0, a1, b):
    return pl.pallas_call(
        _final_body,
        grid=(N // MB,),
        in_specs=[
            pl.BlockSpec((MB, D), lambda i: (i, R)),
            pl.BlockSpec((MB, D), lambda i: (i, 0)),
            pl.BlockSpec((MB, D), lambda i: (i, 0)),
            pl.BlockSpec((1, D), lambda i: (0, 0)),
        ],
        out_specs=pl.BlockSpec((MB, D), lambda i: (i, 0)),
        out_shape=jax.ShapeDtypeStruct((N, D), jnp.float32),
    )(yprev, a0, a1, b)


def kernel(num_x, x, edge_index, edge_type, params):
    src = edge_index[0].astype(jnp.int32)
    dst = edge_index[1].astype(jnp.int32)
    t = edge_type.astype(jnp.int32)
    g = src * RP1 + t

    zbig = jnp.zeros((N, D), jnp.float32)

    s = _get_scale_call()(dst, t)

    bases_l = []
    comp_l = []
    wfulls = []
    for l in range(7):
        pp = params['conv%d' % l]
        b = pp['bases']
        dout = b.shape[2]
        if dout < 128:
            b = jnp.pad(b, ((0, 0), (0, 0), (0, 128 - dout)))
        bases_l.append(b.reshape(b.shape[0], D * 128))
        comp_l.append(pp['comp'])
    w7 = _wprep(jnp.stack(comp_l), jnp.stack(bases_l))
    for l in range(7):
        pp = params['conv%d' % l]
        dout = pp['root'].shape[1]
        rootp = pp['root']
        if dout < 128:
            rootp = jnp.pad(rootp, ((0, 0), (0, 128 - dout)))
        wrel = jnp.transpose(w7[l].reshape(R, D, 128), (1, 0, 2)).reshape(D, R * 128)
        wfulls.append(jnp.concatenate([wrel, rootp], axis=1))

    def pad128(v):
        return jnp.pad(v, (0, 128 - v.shape[0]))[None] if v.shape[0] < 128 else v[None]

    y = _mm_first(num_x, x,
                  params['num_lin_w'], params['num_lin_b'][None],
                  params['prelu_lin'][None], wfulls[0])
    for l in range(1, 7):
        part = _get_agg_call()(y.reshape(N * RP1, D), g, dst, s, zbig)
        y = _mm_mid(y, part[0], part[1],
                    pad128(params['conv%d' % (l - 1)]['bias']),
                    pad128(params['prelu%d' % (l - 1)]), wfulls[l])
    part = _get_agg_call()(y.reshape(N * RP1, D), g, dst, s, zbig)
    out = _final(y, part[0], part[1], pad128(params['conv6']['bias']))
    return out[:, :3]
